# Initial kernel scaffold; baseline (speedup 1.0000x reference)
import jax
import jax.numpy as jnp
from jax.experimental import pallas as pl


def kernel(prediction, ground, trailing_pred, trailing_ground):
    raise NotImplementedError("write your pallas kernel here")


# speedup vs baseline: 94.6834x; 94.6834x over previous
import jax
import jax.numpy as jnp
from jax.experimental import pallas as pl


def _loss_kernel(p_ref, g_ref, tp_ref, out_ref):
    thresh = jnp.max(tp_ref[...])
    p = p_ref[...]
    g = g_ref[...]
    bce = g * jnp.log(p) + (1.0 - g) * jnp.log(1.0 - p)
    fltr = jnp.logical_and(p > thresh, g == 0.0)
    loss = jnp.where(fltr, bce * 20.0, bce)
    out_ref[...] = (jnp.sum(loss) / p.size).reshape(1, 1)


def kernel(prediction, ground, trailing_pred, trailing_ground):
    n = prediction.shape[0]
    m = trailing_pred.shape[0]
    p2 = prediction.reshape(n // 128, 128)
    g2 = ground.reshape(n // 128, 128)
    tp2 = trailing_pred.reshape(m // 128, 128)
    out = pl.pallas_call(
        _loss_kernel,
        out_shape=jax.ShapeDtypeStruct((1, 1), jnp.float32),
    )(p2, g2, tp2)
    return out[0, 0]
